# Initial kernel scaffold; baseline (speedup 1.0000x reference)
import functools

import jax
import jax.numpy as jnp
from jax import lax
from jax.experimental import pallas as pl
from jax.experimental.pallas import tpu as pltpu
from jax.experimental.pallas import tpu_sc as plsc

N = 10000
E = 320000
D = 128
DIM = 8
EPS = 1e-15

NC = 2
NS = 16
NW = NC * NS
CHUNK = 80
RPT = E // NW // CHUNK
NPT = N // NS
ECOLS = E // CHUNK
ROWB = 1000

_mesh = plsc.VectorSubcoreMesh(
    core_axis_name="c", subcore_axis_name="s", num_cores=NC, num_subcores=NS)


def _dot(a, b):
    return lax.dot_general(
        a, b, (((1,), (0,)), ((), ())),
        preferred_element_type=jnp.float32, precision=lax.Precision.HIGHEST)


def _sc_body(with_cnt, *refs):
    if with_cnt:
        (src_h, dst_h, w_h, xl_h, p_h, cnt_h,
         src_v, dst_v, w_v, rows_v, zbuf_v, acc_sh,
         ones_v, zcnt_v, cnt_sh) = refs
    else:
        (src_h, dst_h, w_h, xl_h, p_h,
         src_v, dst_v, w_v, rows_v, zbuf_v, acc_sh) = refs

    cid = lax.axis_index("c")
    sid = lax.axis_index("s")
    wid = cid * NS + sid
    row0 = wid * RPT
    nrow0 = sid * NPT

    pltpu.sync_copy(src_h.at[pl.ds(row0, RPT)], src_v)
    pltpu.sync_copy(dst_h.at[pl.ds(row0, RPT)], dst_v)
    pltpu.sync_copy(w_h.at[pl.ds(row0, RPT)], w_v)

    z16 = jnp.zeros((16,), jnp.float32)

    def zfill(r, _):
        for k in range(D // 16):
            zbuf_v[r, pl.ds(k * 16, 16)] = z16
        return 0
    lax.fori_loop(0, RPT, zfill, 0)
    for q in range(NPT // RPT):
        pltpu.sync_copy(zbuf_v, acc_sh.at[pl.ds(nrow0 + q * RPT, RPT)])

    if with_cnt:
        one16 = jnp.ones((16,), jnp.float32)

        def ofill(r, _):
            ones_v[r, :] = one16
            return 0
        lax.fori_loop(0, CHUNK, ofill, 0)

        def zcfill(r, _):
            zcnt_v[r, :] = z16
            return 0
        lax.fori_loop(0, RPT, zcfill, 0)
        for q in range(NPT // RPT):
            pltpu.sync_copy(zcnt_v, cnt_sh.at[pl.ds(nrow0 + q * RPT, RPT)])

    plsc.subcore_barrier()

    def chunk(j, _):
        pltpu.sync_copy(xl_h.at[src_v.at[j]], rows_v)

        def scale(e, _):
            wv = w_v[j, e]
            for k in range(D // 16):
                sl = pl.ds(k * 16, 16)
                rows_v[e, sl] = rows_v[e, sl] * wv
            return 0
        lax.fori_loop(0, CHUNK, scale, 0)

        pltpu.sync_copy(rows_v, acc_sh.at[dst_v.at[j]], add=True)
        if with_cnt:
            pltpu.sync_copy(ones_v, cnt_sh.at[dst_v.at[j]], add=True)
        return 0
    lax.fori_loop(0, RPT, chunk, 0)

    plsc.subcore_barrier()

    pltpu.sync_copy(acc_sh.at[pl.ds(nrow0, NPT)],
                    p_h.at[cid, pl.ds(nrow0, NPT)])
    if with_cnt:
        pltpu.sync_copy(cnt_sh.at[pl.ds(nrow0, NPT)],
                        cnt_h.at[cid, pl.ds(nrow0, NPT)])


def _make_sc(with_cnt):
    out_type = [jax.ShapeDtypeStruct((NC, N, D), jnp.float32)]
    scratch = [
        pltpu.VMEM((RPT, CHUNK), jnp.int32),
        pltpu.VMEM((RPT, CHUNK), jnp.int32),
        pltpu.VMEM((RPT, CHUNK), jnp.float32),
        pltpu.VMEM((CHUNK, D), jnp.float32),
        pltpu.VMEM((RPT, D), jnp.float32),
        pltpu.VMEM_SHARED((N, D), jnp.float32),
    ]
    if with_cnt:
        out_type.append(jax.ShapeDtypeStruct((NC, N, 16), jnp.float32))
        scratch += [
            pltpu.VMEM((CHUNK, 16), jnp.float32),
            pltpu.VMEM((RPT, 16), jnp.float32),
            pltpu.VMEM_SHARED((N, 16), jnp.float32),
        ]
    return pl.kernel(
        functools.partial(_sc_body, with_cnt),
        out_type=tuple(out_type) if with_cnt else out_type[0],
        mesh=_mesh,
        scratch_types=scratch,
    )


_sc_cnt = _make_sc(True)
_sc_nocnt = _make_sc(False)


def _tw_body(ea, mt1, gt1, mt2, gt2, P, w1, w2):
    e = ea[...]
    p = P[...]
    d1 = e - mt1[...]
    w1[...] = jnp.exp(_dot(d1 * d1 * gt1[...], p))
    d2 = e - mt2[...]
    w2[...] = jnp.exp(_dot(d2 * d2 * gt2[...], p))


def _edge_weights(ea, mt1, gt1, mt2, gt2, P):
    rows = E // 16
    grid = rows // ROWB
    return pl.pallas_call(
        _tw_body,
        grid=(grid,),
        in_specs=[
            pl.BlockSpec((ROWB, 128), lambda i: (i, 0)),
            pl.BlockSpec((1, 128), lambda i: (0, 0)),
            pl.BlockSpec((1, 128), lambda i: (0, 0)),
            pl.BlockSpec((1, 128), lambda i: (0, 0)),
            pl.BlockSpec((1, 128), lambda i: (0, 0)),
            pl.BlockSpec((128, 16), lambda i: (0, 0)),
        ],
        out_specs=[
            pl.BlockSpec((ROWB, 16), lambda i: (i, 0)),
            pl.BlockSpec((ROWB, 16), lambda i: (i, 0)),
        ],
        out_shape=[
            jax.ShapeDtypeStruct((rows, 16), jnp.float32),
            jax.ShapeDtypeStruct((rows, 16), jnp.float32),
        ],
    )(ea, mt1, gt1, mt2, gt2, P)


def _mm2_body(x, a, b, o1, o2):
    xv = x[...]
    o1[...] = _dot(xv, a[...])
    o2[...] = _dot(xv, b[...])


def _mm2(x, a, b):
    return pl.pallas_call(
        _mm2_body,
        grid=(N // ROWB,),
        in_specs=[
            pl.BlockSpec((ROWB, D), lambda i: (i, 0)),
            pl.BlockSpec((D, D), lambda i: (0, 0)),
            pl.BlockSpec((D, D), lambda i: (0, 0)),
        ],
        out_specs=[
            pl.BlockSpec((ROWB, D), lambda i: (i, 0)),
            pl.BlockSpec((ROWB, D), lambda i: (i, 0)),
        ],
        out_shape=[
            jax.ShapeDtypeStruct((N, D), jnp.float32),
            jax.ShapeDtypeStruct((N, D), jnp.float32),
        ],
    )(x, a, b)


def _comb_body(p, c16, r, bvec, g, root, xl2, r2):
    agg = p[0] + p[1]
    c = c16[0, :, 0:1] + c16[1, :, 0:1]
    h = agg / jnp.maximum(c, 1.0) + r[...] + bvec[...]
    xl2[...] = _dot(h, g[...])
    r2[...] = _dot(h, root[...])


def _combine_project(p1, c16, r1, b1, g2, root2):
    return pl.pallas_call(
        _comb_body,
        grid=(N // ROWB,),
        in_specs=[
            pl.BlockSpec((NC, ROWB, D), lambda i: (0, i, 0)),
            pl.BlockSpec((NC, ROWB, 16), lambda i: (0, i, 0)),
            pl.BlockSpec((ROWB, D), lambda i: (i, 0)),
            pl.BlockSpec((1, D), lambda i: (0, 0)),
            pl.BlockSpec((D, D), lambda i: (0, 0)),
            pl.BlockSpec((D, D), lambda i: (0, 0)),
        ],
        out_specs=[
            pl.BlockSpec((ROWB, D), lambda i: (i, 0)),
            pl.BlockSpec((ROWB, D), lambda i: (i, 0)),
        ],
        out_shape=[
            jax.ShapeDtypeStruct((N, D), jnp.float32),
            jax.ShapeDtypeStruct((N, D), jnp.float32),
        ],
    )(p1, c16, r1, b1, g2, root2)


def _final_body(p, c16, r2, bvec, o):
    agg = p[0] + p[1]
    c = c16[0, :, 0:1] + c16[1, :, 0:1]
    o[...] = agg / jnp.maximum(c, 1.0) + r2[...] + bvec[...]


def _final(p2, c16, r2, b2):
    return pl.pallas_call(
        _final_body,
        grid=(N // ROWB,),
        in_specs=[
            pl.BlockSpec((NC, ROWB, D), lambda i: (0, i, 0)),
            pl.BlockSpec((NC, ROWB, 16), lambda i: (0, i, 0)),
            pl.BlockSpec((ROWB, D), lambda i: (i, 0)),
            pl.BlockSpec((1, D), lambda i: (0, 0)),
        ],
        out_specs=pl.BlockSpec((ROWB, D), lambda i: (i, 0)),
        out_shape=jax.ShapeDtypeStruct((N, D), jnp.float32),
    )(p2, c16, r2, b2)


@jax.jit
def kernel(edge_index, edge_weight, x, g1, mu1, sigma1, root1, b1,
           g2, mu2, sigma2, root2, b2):
    src2 = edge_index[0].reshape(ECOLS, CHUNK)
    dst2 = edge_index[1].reshape(ECOLS, CHUNK)
    ea = edge_weight.reshape(E // 16, 128)

    mt1 = jnp.tile(mu1.reshape(-1), 16).reshape(1, 128)
    gt1 = jnp.tile(-0.5 / (EPS + sigma1.reshape(-1) ** 2), 16).reshape(1, 128)
    mt2 = jnp.tile(mu2.reshape(-1), 16).reshape(1, 128)
    gt2 = jnp.tile(-0.5 / (EPS + sigma2.reshape(-1) ** 2), 16).reshape(1, 128)
    P = (jnp.arange(128)[:, None] // DIM == jnp.arange(16)[None, :]
         ).astype(jnp.float32)

    w1f, w2f = _edge_weights(ea, mt1, gt1, mt2, gt2, P)
    w1r = w1f.reshape(ECOLS, CHUNK)
    w2r = w2f.reshape(ECOLS, CHUNK)

    xl1, r1 = _mm2(x, g1, root1)
    p1, c16 = _sc_cnt(src2, dst2, w1r, xl1)
    xl2, r2 = _combine_project(p1, c16, r1, b1.reshape(1, D), g2, root2)
    p2 = _sc_nocnt(src2, dst2, w2r, xl2)
    return _final(p2, c16, r2, b2.reshape(1, D))


# speedup vs baseline: 3.4605x; 3.4605x over previous
import functools

import jax
import jax.numpy as jnp
from jax import lax
from jax.experimental import pallas as pl
from jax.experimental.pallas import tpu as pltpu
from jax.experimental.pallas import tpu_sc as plsc

N = 10000
E = 320000
D = 128
DIM = 8
EPS = 1e-15

NC = 2
NS = 16
NW = NC * NS
CHUNK = 80
RPT = E // NW // CHUNK
ROWB = 1000

@functools.lru_cache(maxsize=None)
def _get_mesh():
    return plsc.VectorSubcoreMesh(
        core_axis_name="c", subcore_axis_name="s",
        num_cores=NC, num_subcores=NS)


def _dot(a, b):
    return lax.dot_general(
        a, b, (((1,), (0,)), ((), ())),
        preferred_element_type=jnp.float32, precision=lax.Precision.HIGHEST)


SLAB = 624
LSLAB = N - (NS - 1) * SLAB


def _zero_slab(sid, rows_v, acc_sh):
    @pl.when(sid < NS - 1)
    def _():
        base = sid * SLAB
        for q in range(SLAB // CHUNK):
            pltpu.sync_copy(rows_v, acc_sh.at[pl.ds(base + q * CHUNK, CHUNK)])
        rem = SLAB % CHUNK
        pltpu.sync_copy(rows_v.at[pl.ds(0, rem)],
                        acc_sh.at[pl.ds(base + SLAB - rem, rem)])

    @pl.when(sid == NS - 1)
    def _():
        base = (NS - 1) * SLAB
        for q in range(LSLAB // CHUNK):
            pltpu.sync_copy(rows_v, acc_sh.at[pl.ds(base + q * CHUNK, CHUNK)])


def _emit_slab(cid, sid, acc_sh, p_h):
    @pl.when(sid < NS - 1)
    def _():
        base = sid * SLAB
        pltpu.sync_copy(acc_sh.at[pl.ds(base, SLAB)],
                        p_h.at[cid, pl.ds(base, SLAB)])

    @pl.when(sid == NS - 1)
    def _():
        base = (NS - 1) * SLAB
        pltpu.sync_copy(acc_sh.at[pl.ds(base, LSLAB)],
                        p_h.at[cid, pl.ds(base, LSLAB)])


def _sc_body(src_h, dst_h, w_h, xl_h, p_h,
             src_v, dst_v, w_v, rows_v, acc_sh):
    cid = lax.axis_index("c")
    sid = lax.axis_index("s")
    wid = cid * NS + sid

    z16 = jnp.zeros((16,), jnp.float32)

    def zfill(r, _):
        for k in range(D // 16):
            rows_v[r, pl.ds(k * 16, 16)] = z16
        return 0
    lax.fori_loop(0, CHUNK, zfill, 0)

    _zero_slab(sid, rows_v, acc_sh)
    plsc.subcore_barrier()

    def chunk(j, _):
        pltpu.sync_copy(src_h.at[wid, j], src_v)
        pltpu.sync_copy(dst_h.at[wid, j], dst_v)
        pltpu.sync_copy(w_h.at[wid, j], w_v)
        pltpu.sync_copy(xl_h.at[src_v], rows_v)

        for g in range(CHUNK // 16):
            wvec = w_v[pl.ds(g * 16, 16)]
            for i in range(16):
                wv = wvec[i]
                e = g * 16 + i
                for k in range(D // 16):
                    sl = pl.ds(k * 16, 16)
                    rows_v[e, sl] = rows_v[e, sl] * wv

        pltpu.sync_copy(rows_v, acc_sh.at[dst_v], add=True)
        return 0
    lax.fori_loop(0, RPT, chunk, 0)

    plsc.subcore_barrier()
    _emit_slab(cid, sid, acc_sh, p_h)


def _cnt_body(dst_h, cnt_out, dst_v, ones_v, zr_v, cnt_sh):
    cid = lax.axis_index("c")
    sid = lax.axis_index("s")
    wid = cid * NS + sid

    z16 = jnp.zeros((16,), jnp.float32)
    one16 = jnp.ones((16,), jnp.float32)

    def fill(r, _):
        for k in range(D // 16):
            ones_v[r, pl.ds(k * 16, 16)] = one16
            zr_v[r, pl.ds(k * 16, 16)] = z16
        return 0
    lax.fori_loop(0, CHUNK, fill, 0)

    _zero_slab(sid, zr_v, cnt_sh)
    plsc.subcore_barrier()

    def chunk(j, _):
        pltpu.sync_copy(dst_h.at[wid, j], dst_v)
        pltpu.sync_copy(ones_v, cnt_sh.at[dst_v], add=True)
        return 0
    lax.fori_loop(0, RPT, chunk, 0)

    plsc.subcore_barrier()
    _emit_slab(cid, sid, cnt_sh, cnt_out)


@functools.lru_cache(maxsize=None)
def _make_sc():
    return pl.kernel(
        _sc_body,
        out_type=jax.ShapeDtypeStruct((NC, N, D), jnp.float32),
        mesh=_get_mesh(),
        scratch_types=[
            pltpu.VMEM((CHUNK,), jnp.int32),
            pltpu.VMEM((CHUNK,), jnp.int32),
            pltpu.VMEM((CHUNK,), jnp.float32),
            pltpu.VMEM((CHUNK, D), jnp.float32),
            pltpu.VMEM_SHARED((N, D), jnp.float32),
        ],
    )


@functools.lru_cache(maxsize=None)
def _make_cnt():
    return pl.kernel(
        _cnt_body,
        out_type=jax.ShapeDtypeStruct((NC, N, D), jnp.float32),
        mesh=_get_mesh(),
        scratch_types=[
            pltpu.VMEM((CHUNK,), jnp.int32),
            pltpu.VMEM((CHUNK, D), jnp.float32),
            pltpu.VMEM((CHUNK, D), jnp.float32),
            pltpu.VMEM_SHARED((N, D), jnp.float32),
        ],
    )


def _tw_body(ea, mt1, gt1, mt2, gt2, P, w1, w2):
    e = ea[...]
    p = P[...]
    d1 = e - mt1[...]
    w1[...] = jnp.exp(_dot(d1 * d1 * gt1[...], p))
    d2 = e - mt2[...]
    w2[...] = jnp.exp(_dot(d2 * d2 * gt2[...], p))


def _edge_weights(ea, mt1, gt1, mt2, gt2, P):
    rows = E // 16
    grid = rows // ROWB
    return pl.pallas_call(
        _tw_body,
        grid=(grid,),
        in_specs=[
            pl.BlockSpec((ROWB, 128), lambda i: (i, 0)),
            pl.BlockSpec((1, 128), lambda i: (0, 0)),
            pl.BlockSpec((1, 128), lambda i: (0, 0)),
            pl.BlockSpec((1, 128), lambda i: (0, 0)),
            pl.BlockSpec((1, 128), lambda i: (0, 0)),
            pl.BlockSpec((128, 16), lambda i: (0, 0)),
        ],
        out_specs=[
            pl.BlockSpec((ROWB, 16), lambda i: (i, 0)),
            pl.BlockSpec((ROWB, 16), lambda i: (i, 0)),
        ],
        out_shape=[
            jax.ShapeDtypeStruct((rows, 16), jnp.float32),
            jax.ShapeDtypeStruct((rows, 16), jnp.float32),
        ],
    )(ea, mt1, gt1, mt2, gt2, P)


def _mm2_body(x, a, b, o1, o2):
    xv = x[...]
    o1[...] = _dot(xv, a[...])
    o2[...] = _dot(xv, b[...])


def _mm2(x, a, b):
    return pl.pallas_call(
        _mm2_body,
        grid=(N // ROWB,),
        in_specs=[
            pl.BlockSpec((ROWB, D), lambda i: (i, 0)),
            pl.BlockSpec((D, D), lambda i: (0, 0)),
            pl.BlockSpec((D, D), lambda i: (0, 0)),
        ],
        out_specs=[
            pl.BlockSpec((ROWB, D), lambda i: (i, 0)),
            pl.BlockSpec((ROWB, D), lambda i: (i, 0)),
        ],
        out_shape=[
            jax.ShapeDtypeStruct((N, D), jnp.float32),
            jax.ShapeDtypeStruct((N, D), jnp.float32),
        ],
    )(x, a, b)


def _comb_body(p, cnt, r, bvec, g, root, xl2, r2, cinv):
    agg = p[0] + p[1]
    c = cnt[0, :, 0:1] + cnt[1, :, 0:1]
    ci = 1.0 / jnp.maximum(c, 1.0)
    h = agg * ci + r[...] + bvec[...]
    xl2[...] = _dot(h, g[...])
    r2[...] = _dot(h, root[...])
    cinv[...] = jnp.broadcast_to(ci, (h.shape[0], D))


def _combine_project(p1, cnt, r1, b1, g2, root2):
    return pl.pallas_call(
        _comb_body,
        grid=(N // ROWB,),
        in_specs=[
            pl.BlockSpec((NC, ROWB, D), lambda i: (0, i, 0)),
            pl.BlockSpec((NC, ROWB, D), lambda i: (0, i, 0)),
            pl.BlockSpec((ROWB, D), lambda i: (i, 0)),
            pl.BlockSpec((1, D), lambda i: (0, 0)),
            pl.BlockSpec((D, D), lambda i: (0, 0)),
            pl.BlockSpec((D, D), lambda i: (0, 0)),
        ],
        out_specs=[
            pl.BlockSpec((ROWB, D), lambda i: (i, 0)),
            pl.BlockSpec((ROWB, D), lambda i: (i, 0)),
            pl.BlockSpec((ROWB, D), lambda i: (i, 0)),
        ],
        out_shape=[
            jax.ShapeDtypeStruct((N, D), jnp.float32),
            jax.ShapeDtypeStruct((N, D), jnp.float32),
            jax.ShapeDtypeStruct((N, D), jnp.float32),
        ],
    )(p1, cnt, r1, b1, g2, root2)


def _final_body(p, cinv, r2, bvec, o):
    agg = p[0] + p[1]
    o[...] = agg * cinv[...] + r2[...] + bvec[...]


def _final(p2, cinv, r2, b2):
    return pl.pallas_call(
        _final_body,
        grid=(N // ROWB,),
        in_specs=[
            pl.BlockSpec((NC, ROWB, D), lambda i: (0, i, 0)),
            pl.BlockSpec((ROWB, D), lambda i: (i, 0)),
            pl.BlockSpec((ROWB, D), lambda i: (i, 0)),
            pl.BlockSpec((1, D), lambda i: (0, 0)),
        ],
        out_specs=pl.BlockSpec((ROWB, D), lambda i: (i, 0)),
        out_shape=jax.ShapeDtypeStruct((N, D), jnp.float32),
    )(p2, cinv, r2, b2)


@jax.jit
def kernel(edge_index, edge_weight, x, g1, mu1, sigma1, root1, b1,
           g2, mu2, sigma2, root2, b2):
    src2 = edge_index[0].reshape(NW, RPT, CHUNK)
    dst2 = edge_index[1].reshape(NW, RPT, CHUNK)
    ea = edge_weight.reshape(E // 16, 128)

    mt1 = jnp.tile(mu1.reshape(-1), 16).reshape(1, 128)
    gt1 = jnp.tile(-0.5 / (EPS + sigma1.reshape(-1) ** 2), 16).reshape(1, 128)
    mt2 = jnp.tile(mu2.reshape(-1), 16).reshape(1, 128)
    gt2 = jnp.tile(-0.5 / (EPS + sigma2.reshape(-1) ** 2), 16).reshape(1, 128)
    P = (jnp.arange(128)[:, None] // DIM == jnp.arange(16)[None, :]
         ).astype(jnp.float32)

    w1f, w2f = _edge_weights(ea, mt1, gt1, mt2, gt2, P)
    w1r = w1f.reshape(NW, RPT, CHUNK)
    w2r = w2f.reshape(NW, RPT, CHUNK)

    cnt = _make_cnt()(dst2)
    xl1, r1 = _mm2(x, g1, root1)
    p1 = _make_sc()(src2, dst2, w1r, xl1)
    xl2, r2, cinv = _combine_project(p1, cnt, r1, b1.reshape(1, D), g2, root2)
    p2 = _make_sc()(src2, dst2, w2r, xl2)
    return _final(p2, cinv, r2, b2.reshape(1, D))


# speedup vs baseline: 4.3387x; 1.2538x over previous
import functools

import jax
import jax.numpy as jnp
from jax import lax
from jax.experimental import pallas as pl
from jax.experimental.pallas import tpu as pltpu
from jax.experimental.pallas import tpu_sc as plsc

N = 10000
E = 320000
D = 128
DIM = 8
EPS = 1e-15

NC = 2
NS = 16
NW = NC * NS
CHUNK = 80
RPT = E // NW // CHUNK
ROWB = 1000

@functools.lru_cache(maxsize=None)
def _get_mesh():
    return plsc.VectorSubcoreMesh(
        core_axis_name="c", subcore_axis_name="s",
        num_cores=NC, num_subcores=NS)


def _dot(a, b):
    return lax.dot_general(
        a, b, (((1,), (0,)), ((), ())),
        preferred_element_type=jnp.float32, precision=lax.Precision.HIGHEST)


SLAB = 624
LSLAB = N - (NS - 1) * SLAB


def _zero_slab(sid, rows_v, acc_sh):
    @pl.when(sid < NS - 1)
    def _():
        base = sid * SLAB
        for q in range(SLAB // CHUNK):
            pltpu.sync_copy(rows_v, acc_sh.at[pl.ds(base + q * CHUNK, CHUNK)])
        rem = SLAB % CHUNK
        pltpu.sync_copy(rows_v.at[pl.ds(0, rem)],
                        acc_sh.at[pl.ds(base + SLAB - rem, rem)])

    @pl.when(sid == NS - 1)
    def _():
        base = (NS - 1) * SLAB
        for q in range(LSLAB // CHUNK):
            pltpu.sync_copy(rows_v, acc_sh.at[pl.ds(base + q * CHUNK, CHUNK)])


def _emit_slab(cid, sid, acc_sh, p_h):
    @pl.when(sid < NS - 1)
    def _():
        base = sid * SLAB
        pltpu.sync_copy(acc_sh.at[pl.ds(base, SLAB)],
                        p_h.at[cid, pl.ds(base, SLAB)])

    @pl.when(sid == NS - 1)
    def _():
        base = (NS - 1) * SLAB
        pltpu.sync_copy(acc_sh.at[pl.ds(base, LSLAB)],
                        p_h.at[cid, pl.ds(base, LSLAB)])


def _sc_body(src_h, dst_h, w_h, xl_h, p_h,
             dst_v, rows_v, srcs_v, ws_v, acc_sh):
    cid = lax.axis_index("c")
    sid = lax.axis_index("s")
    wid = cid * NS + sid

    pltpu.sync_copy(src_h.at[wid], srcs_v)
    pltpu.sync_copy(w_h.at[wid], ws_v)

    z16 = jnp.zeros((16,), jnp.float32)

    def zfill(r, _):
        for k in range(D // 16):
            rows_v[r, pl.ds(k * 16, 16)] = z16
        return 0
    lax.fori_loop(0, CHUNK, zfill, 0)

    _zero_slab(sid, rows_v, acc_sh)
    plsc.subcore_barrier()

    def chunk(j, _):
        pltpu.sync_copy(dst_h.at[wid, j], dst_v)
        pltpu.sync_copy(xl_h.at[srcs_v.at[j]], rows_v)

        for g in range(CHUNK // 16):
            wvec = ws_v[j, pl.ds(g * 16, 16)]
            for i in range(16):
                wv = wvec[i]
                e = g * 16 + i
                for k in range(D // 16):
                    sl = pl.ds(k * 16, 16)
                    rows_v[e, sl] = rows_v[e, sl] * wv

        pltpu.sync_copy(rows_v, acc_sh.at[dst_v], add=True)
        return 0
    lax.fori_loop(0, RPT, chunk, 0)

    plsc.subcore_barrier()
    _emit_slab(cid, sid, acc_sh, p_h)


def _cnt_body(dst_h, cnt_out, dst_v, ones_v, zr_v, cnt_sh):
    cid = lax.axis_index("c")
    sid = lax.axis_index("s")
    wid = cid * NS + sid

    z16 = jnp.zeros((16,), jnp.float32)
    one16 = jnp.ones((16,), jnp.float32)

    def fill(r, _):
        for k in range(D // 16):
            ones_v[r, pl.ds(k * 16, 16)] = one16
            zr_v[r, pl.ds(k * 16, 16)] = z16
        return 0
    lax.fori_loop(0, CHUNK, fill, 0)

    _zero_slab(sid, zr_v, cnt_sh)
    plsc.subcore_barrier()

    def chunk(j, _):
        pltpu.sync_copy(dst_h.at[wid, j], dst_v)
        pltpu.sync_copy(ones_v, cnt_sh.at[dst_v], add=True)
        return 0
    lax.fori_loop(0, RPT, chunk, 0)

    plsc.subcore_barrier()
    _emit_slab(cid, sid, cnt_sh, cnt_out)


@functools.lru_cache(maxsize=None)
def _make_sc():
    return pl.kernel(
        _sc_body,
        out_type=jax.ShapeDtypeStruct((NC, N, D), jnp.float32),
        mesh=_get_mesh(),
        scratch_types=[
            pltpu.VMEM((CHUNK,), jnp.int32),
            pltpu.VMEM((CHUNK, D), jnp.float32),
            pltpu.VMEM((RPT, CHUNK), jnp.int32),
            pltpu.VMEM((RPT, CHUNK), jnp.float32),
            pltpu.VMEM_SHARED((N, D), jnp.float32),
        ],
    )


@functools.lru_cache(maxsize=None)
def _make_cnt():
    return pl.kernel(
        _cnt_body,
        out_type=jax.ShapeDtypeStruct((NC, N, D), jnp.float32),
        mesh=_get_mesh(),
        scratch_types=[
            pltpu.VMEM((CHUNK,), jnp.int32),
            pltpu.VMEM((CHUNK, D), jnp.float32),
            pltpu.VMEM((CHUNK, D), jnp.float32),
            pltpu.VMEM_SHARED((N, D), jnp.float32),
        ],
    )


def _tw_body(ea, mt1, gt1, mt2, gt2, P, w1, w2):
    e = ea[...]
    p = P[...]
    d1 = e - mt1[...]
    w1[...] = jnp.exp(_dot(d1 * d1 * gt1[...], p))
    d2 = e - mt2[...]
    w2[...] = jnp.exp(_dot(d2 * d2 * gt2[...], p))


def _edge_weights(ea, mt1, gt1, mt2, gt2, P):
    rows = E // 16
    grid = rows // ROWB
    return pl.pallas_call(
        _tw_body,
        grid=(grid,),
        in_specs=[
            pl.BlockSpec((ROWB, 128), lambda i: (i, 0)),
            pl.BlockSpec((1, 128), lambda i: (0, 0)),
            pl.BlockSpec((1, 128), lambda i: (0, 0)),
            pl.BlockSpec((1, 128), lambda i: (0, 0)),
            pl.BlockSpec((1, 128), lambda i: (0, 0)),
            pl.BlockSpec((128, 16), lambda i: (0, 0)),
        ],
        out_specs=[
            pl.BlockSpec((ROWB, 16), lambda i: (i, 0)),
            pl.BlockSpec((ROWB, 16), lambda i: (i, 0)),
        ],
        out_shape=[
            jax.ShapeDtypeStruct((rows, 16), jnp.float32),
            jax.ShapeDtypeStruct((rows, 16), jnp.float32),
        ],
    )(ea, mt1, gt1, mt2, gt2, P)


def _mm2_body(x, a, b, o1, o2):
    xv = x[...]
    o1[...] = _dot(xv, a[...])
    o2[...] = _dot(xv, b[...])


def _mm2(x, a, b):
    return pl.pallas_call(
        _mm2_body,
        grid=(N // ROWB,),
        in_specs=[
            pl.BlockSpec((ROWB, D), lambda i: (i, 0)),
            pl.BlockSpec((D, D), lambda i: (0, 0)),
            pl.BlockSpec((D, D), lambda i: (0, 0)),
        ],
        out_specs=[
            pl.BlockSpec((ROWB, D), lambda i: (i, 0)),
            pl.BlockSpec((ROWB, D), lambda i: (i, 0)),
        ],
        out_shape=[
            jax.ShapeDtypeStruct((N, D), jnp.float32),
            jax.ShapeDtypeStruct((N, D), jnp.float32),
        ],
    )(x, a, b)


def _comb_body(p, cnt, r, bvec, g, root, xl2, r2, cinv):
    agg = p[0] + p[1]
    c = cnt[0, :, 0:1] + cnt[1, :, 0:1]
    ci = 1.0 / jnp.maximum(c, 1.0)
    h = agg * ci + r[...] + bvec[...]
    xl2[...] = _dot(h, g[...])
    r2[...] = _dot(h, root[...])
    cinv[...] = jnp.broadcast_to(ci, (h.shape[0], D))


def _combine_project(p1, cnt, r1, b1, g2, root2):
    return pl.pallas_call(
        _comb_body,
        grid=(N // ROWB,),
        in_specs=[
            pl.BlockSpec((NC, ROWB, D), lambda i: (0, i, 0)),
            pl.BlockSpec((NC, ROWB, D), lambda i: (0, i, 0)),
            pl.BlockSpec((ROWB, D), lambda i: (i, 0)),
            pl.BlockSpec((1, D), lambda i: (0, 0)),
            pl.BlockSpec((D, D), lambda i: (0, 0)),
            pl.BlockSpec((D, D), lambda i: (0, 0)),
        ],
        out_specs=[
            pl.BlockSpec((ROWB, D), lambda i: (i, 0)),
            pl.BlockSpec((ROWB, D), lambda i: (i, 0)),
            pl.BlockSpec((ROWB, D), lambda i: (i, 0)),
        ],
        out_shape=[
            jax.ShapeDtypeStruct((N, D), jnp.float32),
            jax.ShapeDtypeStruct((N, D), jnp.float32),
            jax.ShapeDtypeStruct((N, D), jnp.float32),
        ],
    )(p1, cnt, r1, b1, g2, root2)


def _final_body(p, cinv, r2, bvec, o):
    agg = p[0] + p[1]
    o[...] = agg * cinv[...] + r2[...] + bvec[...]


def _final(p2, cinv, r2, b2):
    return pl.pallas_call(
        _final_body,
        grid=(N // ROWB,),
        in_specs=[
            pl.BlockSpec((NC, ROWB, D), lambda i: (0, i, 0)),
            pl.BlockSpec((ROWB, D), lambda i: (i, 0)),
            pl.BlockSpec((ROWB, D), lambda i: (i, 0)),
            pl.BlockSpec((1, D), lambda i: (0, 0)),
        ],
        out_specs=pl.BlockSpec((ROWB, D), lambda i: (i, 0)),
        out_shape=jax.ShapeDtypeStruct((N, D), jnp.float32),
    )(p2, cinv, r2, b2)


@jax.jit
def kernel(edge_index, edge_weight, x, g1, mu1, sigma1, root1, b1,
           g2, mu2, sigma2, root2, b2):
    src2 = edge_index[0].reshape(NW, RPT, CHUNK)
    dst2 = edge_index[1].reshape(NW, RPT, CHUNK)
    ea = edge_weight.reshape(E // 16, 128)

    mt1 = jnp.tile(mu1.reshape(-1), 16).reshape(1, 128)
    gt1 = jnp.tile(-0.5 / (EPS + sigma1.reshape(-1) ** 2), 16).reshape(1, 128)
    mt2 = jnp.tile(mu2.reshape(-1), 16).reshape(1, 128)
    gt2 = jnp.tile(-0.5 / (EPS + sigma2.reshape(-1) ** 2), 16).reshape(1, 128)
    P = (jnp.arange(128)[:, None] // DIM == jnp.arange(16)[None, :]
         ).astype(jnp.float32)

    w1f, w2f = _edge_weights(ea, mt1, gt1, mt2, gt2, P)
    w1r = w1f.reshape(NW, RPT, CHUNK)
    w2r = w2f.reshape(NW, RPT, CHUNK)

    cnt = _make_cnt()(dst2)
    xl1, r1 = _mm2(x, g1, root1)
    p1 = _make_sc()(src2, dst2, w1r, xl1)
    xl2, r2, cinv = _combine_project(p1, cnt, r1, b1.reshape(1, D), g2, root2)
    p2 = _make_sc()(src2, dst2, w2r, xl2)
    return _final(p2, cinv, r2, b2.reshape(1, D))
